# Initial kernel scaffold; baseline (speedup 1.0000x reference)
import jax
import jax.numpy as jnp
from jax.experimental import pallas as pl


def kernel(X, S, V, Vt, R, W1s, W1l, b1, W2s, W2l, b2, Wm, bm, filt1_W, filt1_b, filt2_W, filt2_b):
    raise NotImplementedError("write your pallas kernel here")


# speedup vs baseline: 1.1885x; 1.1885x over previous
import functools

import jax
import jax.numpy as jnp
from jax import lax
from jax.experimental import pallas as pl
from jax.experimental.pallas import tpu as pltpu

_NCLASS = 64
_LONG = (2, 4)


def _ln_kernel(xt_ref, s_ref, v_ref, vt_ref, d1_ref, d2_ref,
               w1s_ref, w1l_ref, b1_ref,
               w2s_ref, w2l_ref, b2_ref,
               wm_ref, bm_ref, o_ref, *, n_long, nclass):
    bf = jnp.bfloat16
    f32 = jnp.float32
    S = s_ref[...]
    V = v_ref[...]
    Vt = vt_ref[...]

    def layer(At_b, d_ref, ws_ref, wl_ref, b_ref):
        A1 = jnp.dot(At_b, S, preferred_element_type=f32)
        A1b = A1.astype(bf)
        A2 = jnp.dot(A1b, S, preferred_element_type=f32)
        A2b = A2.astype(bf)

        U = jnp.dot(At_b, V, preferred_element_type=f32)
        D = d_ref[...]
        Gt = jnp.concatenate([U * D[t:t + 1, :] for t in range(n_long)],
                             axis=0)
        Pt = jnp.dot(wl_ref[...], Gt.astype(bf),
                     preferred_element_type=f32)

        acc = jnp.dot(ws_ref[0], At_b, preferred_element_type=f32)
        acc = acc + jnp.dot(ws_ref[1], A1b, preferred_element_type=f32)
        acc = acc + jnp.dot(ws_ref[2], A2b, preferred_element_type=f32)
        acc = acc + jnp.dot(Pt.astype(bf), Vt, preferred_element_type=f32)
        return jnp.maximum(acc + b_ref[...], 0.0)

    Z1 = layer(xt_ref[...], d1_ref, w1s_ref, w1l_ref, b1_ref)
    Z2 = layer(Z1.astype(bf), d2_ref, w2s_ref, w2l_ref, b2_ref)

    logits = lax.dot_general(Z2.astype(bf), wm_ref[...],
                             (((0,), (0,)), ((), ())),
                             preferred_element_type=f32) + bm_ref[...]
    logits = jnp.maximum(logits, 0.0)
    col = lax.broadcasted_iota(jnp.int32, logits.shape, 1)
    logits = jnp.where(col < nclass, logits, jnp.float32(-1e30))
    m = jnp.max(logits, axis=1, keepdims=True)
    sh = logits - m
    o_ref[...] = sh - jnp.log(jnp.sum(jnp.exp(sh), axis=1, keepdims=True))


def kernel(X, S, V, Vt, R, W1s, W1l, b1, W2s, W2l, b2, Wm, bm,
           filt1_W, filt1_b, filt2_W, filt2_b):
    G, N, F0 = X.shape
    Kp = V.shape[-1]
    n_long = len(_LONG)
    nclass = _NCLASS
    F1 = b1.shape[1]
    F2 = b2.shape[1]
    Cpad = Wm.shape[1]

    f32 = jnp.float32
    D_raw = jnp.stack([R ** t for t in _LONG], axis=-1)
    D1t = jnp.swapaxes(
        jnp.einsum("gkt,ts->gks", D_raw, filt1_W) + filt1_b, 1, 2).astype(f32)
    D2t = jnp.swapaxes(
        jnp.einsum("gkt,ts->gks", D_raw, filt2_W) + filt2_b, 1, 2).astype(f32)

    Xt = jnp.swapaxes(X, 1, 2)
    W1st = jnp.swapaxes(W1s, 1, 2)
    W2st = jnp.swapaxes(W2s, 1, 2)
    W1lt = W1l.T
    W2lt = W2l.T
    b1t = b1.reshape(F1, 1).astype(f32)
    b2t = b2.reshape(F2, 1).astype(f32)

    kern = functools.partial(_ln_kernel, n_long=n_long, nclass=nclass)

    in_specs = [
        pl.BlockSpec((None, F0, N), lambda g: (g, 0, 0)),
        pl.BlockSpec((None, N, N), lambda g: (g, 0, 0)),
        pl.BlockSpec((None, N, Kp), lambda g: (g, 0, 0)),
        pl.BlockSpec((None, Kp, N), lambda g: (g, 0, 0)),
        pl.BlockSpec((None, n_long, Kp), lambda g: (g, 0, 0)),
        pl.BlockSpec((None, n_long, Kp), lambda g: (g, 0, 0)),
        pl.BlockSpec(W1st.shape, lambda g: (0, 0, 0)),
        pl.BlockSpec(W1lt.shape, lambda g: (0, 0)),
        pl.BlockSpec(b1t.shape, lambda g: (0, 0)),
        pl.BlockSpec(W2st.shape, lambda g: (0, 0, 0)),
        pl.BlockSpec(W2lt.shape, lambda g: (0, 0)),
        pl.BlockSpec(b2t.shape, lambda g: (0, 0)),
        pl.BlockSpec(Wm.shape, lambda g: (0, 0)),
        pl.BlockSpec(bm.shape, lambda g: (0, 0)),
    ]

    flops_layer = G * (2 * 2 * N * N * F1
                       + 2 * 3 * N * F1 * F1
                       + 2 * N * F1 * Kp * 2)
    cost = pl.CostEstimate(
        flops=int(2 * flops_layer + G * 2 * N * F2 * Cpad),
        transcendentals=int(G * N * (Cpad + 1)),
        bytes_accessed=int(Xt.size * 2 + S.size * 2 + V.size * 4
                           + G * N * Cpad * 4))

    out = pl.pallas_call(
        kern,
        out_shape=jax.ShapeDtypeStruct((G, N, Cpad), jnp.float32),
        grid=(G,),
        in_specs=in_specs,
        out_specs=pl.BlockSpec((None, N, Cpad), lambda g: (g, 0, 0)),
        compiler_params=pltpu.CompilerParams(
            dimension_semantics=("parallel",)),
        cost_estimate=cost,
    )(Xt, S, V, Vt, D1t, D2t, W1st, W1lt, b1t, W2st, W2lt, b2t, Wm, bm)
    return out[..., :nclass]


# speedup vs baseline: 1.4418x; 1.2131x over previous
import functools

import jax
import jax.numpy as jnp
from jax import lax
from jax.experimental import pallas as pl
from jax.experimental.pallas import tpu as pltpu

_NCLASS = 64
_LONG = (2, 4)
_GPB = 2


def _ta(lhs, rhs):
    return lax.dot_general(lhs, rhs, (((0,), (0,)), ((), ())),
                           preferred_element_type=jnp.float32)


def _tab(lhs, rhs):
    return lax.dot_general(lhs, rhs, (((0,), (1,)), ((), ())),
                           preferred_element_type=jnp.float32)


def _nn(lhs, rhs):
    return jnp.dot(lhs, rhs, preferred_element_type=jnp.float32)


def _ln_kernel(x_ref, s_ref, v_ref, vt_ref, r_ref,
               w1s_ref, w1l_ref, b1_ref,
               w2s_ref, w2l_ref, b2_ref,
               wm_ref, bm_ref, f1w_ref, f1b_ref, f2w_ref, f2b_ref,
               o_ref, *, n_long, nclass, gpb):
    bf = jnp.bfloat16

    b1c = jnp.transpose(b1_ref[...], (1, 0))
    b2c = jnp.transpose(b2_ref[...], (1, 0))
    bm_row = bm_ref[...]
    f1w, f1b = f1w_ref[...], f1b_ref[...]
    f2w, f2b = f2w_ref[...], f2b_ref[...]

    for g in range(gpb):
        Xg = x_ref[g]
        Sg = s_ref[g]
        Vg = v_ref[g]
        Vtg = vt_ref[g]
        Rv = r_ref[g]

        Rp = {1: Rv}
        cur = Rv
        for t in range(2, max(_LONG) + 1):
            cur = cur * Rv
            if t in _LONG:
                Rp[t] = cur

        def dvec(fw, fb, s):
            d = fb[0:1, s:s + 1]
            for ti, t in enumerate(_LONG):
                d = d + fw[ti:ti + 1, s:s + 1] * Rp[t]
            return d

        A1 = _ta(Xg, Sg)
        A1b = A1.astype(bf)
        A2 = _nn(A1b, Sg)
        A2b = A2.astype(bf)
        U = _ta(Xg, Vg)
        Gt = jnp.concatenate(
            [U * dvec(f1w, f1b, s) for s in range(n_long)], axis=0)
        Pt = _ta(w1l_ref[...], Gt.astype(bf))
        acc = _tab(w1s_ref[0], Xg)
        acc = acc + _ta(w1s_ref[1], A1b)
        acc = acc + _ta(w1s_ref[2], A2b)
        acc = acc + _nn(Pt.astype(bf), Vtg)
        Z1b = jnp.maximum(acc + b1c, 0.0).astype(bf)

        B1 = _nn(Z1b, Sg)
        B1b = B1.astype(bf)
        B2 = _nn(B1b, Sg)
        B2b = B2.astype(bf)
        U2 = _nn(Z1b, Vg)
        Gt2 = jnp.concatenate(
            [U2 * dvec(f2w, f2b, s) for s in range(n_long)], axis=0)
        Pt2 = _ta(w2l_ref[...], Gt2.astype(bf))
        acc2 = _ta(w2s_ref[0], Z1b)
        acc2 = acc2 + _ta(w2s_ref[1], B1b)
        acc2 = acc2 + _ta(w2s_ref[2], B2b)
        acc2 = acc2 + _nn(Pt2.astype(bf), Vtg)
        Z2b = jnp.maximum(acc2 + b2c, 0.0).astype(bf)

        logits = _ta(Z2b, wm_ref[...]) + bm_row
        logits = jnp.maximum(logits, 0.0)
        col = lax.broadcasted_iota(jnp.int32, logits.shape, 1)
        logits = jnp.where(col < nclass, logits, jnp.float32(-1e30))
        m = jnp.max(logits, axis=1, keepdims=True)
        sh = logits - m
        lsm = sh - jnp.log(jnp.sum(jnp.exp(sh), axis=1, keepdims=True))
        o_ref[g] = lsm[:, :nclass]


def kernel(X, S, V, Vt, R, W1s, W1l, b1, W2s, W2l, b2, Wm, bm,
           filt1_W, filt1_b, filt2_W, filt2_b):
    G, N, F0 = X.shape
    Kp = V.shape[-1]
    n_long = len(_LONG)
    nclass = _NCLASS
    F1 = b1.shape[1]
    F2 = b2.shape[1]
    Cpad = Wm.shape[1]
    gpb = _GPB if G % _GPB == 0 else 1

    R3 = R.reshape(G, 1, Kp).astype(jnp.float32)

    kern = functools.partial(_ln_kernel, n_long=n_long, nclass=nclass,
                             gpb=gpb)

    in_specs = [
        pl.BlockSpec((gpb, N, F0), lambda i: (i, 0, 0)),
        pl.BlockSpec((gpb, N, N), lambda i: (i, 0, 0)),
        pl.BlockSpec((gpb, N, Kp), lambda i: (i, 0, 0)),
        pl.BlockSpec((gpb, Kp, N), lambda i: (i, 0, 0)),
        pl.BlockSpec((gpb, 1, Kp), lambda i: (i, 0, 0)),
        pl.BlockSpec(W1s.shape, lambda i: (0, 0, 0)),
        pl.BlockSpec(W1l.shape, lambda i: (0, 0)),
        pl.BlockSpec(b1.shape, lambda i: (0, 0)),
        pl.BlockSpec(W2s.shape, lambda i: (0, 0, 0)),
        pl.BlockSpec(W2l.shape, lambda i: (0, 0)),
        pl.BlockSpec(b2.shape, lambda i: (0, 0)),
        pl.BlockSpec(Wm.shape, lambda i: (0, 0)),
        pl.BlockSpec(bm.shape, lambda i: (0, 0)),
        pl.BlockSpec(filt1_W.shape, lambda i: (0, 0)),
        pl.BlockSpec(filt1_b.shape, lambda i: (0, 0)),
        pl.BlockSpec(filt2_W.shape, lambda i: (0, 0)),
        pl.BlockSpec(filt2_b.shape, lambda i: (0, 0)),
    ]

    flops_layer = G * (2 * 2 * N * N * F1
                       + 2 * 3 * N * F1 * F1
                       + 2 * N * F1 * Kp * 2)
    cost = pl.CostEstimate(
        flops=int(2 * flops_layer + G * 2 * N * F2 * Cpad),
        transcendentals=int(G * N * (Cpad + 1)),
        bytes_accessed=int(X.size * 2 + S.size * 2 + 2 * V.size * 2
                           + G * N * nclass * 4))

    return pl.pallas_call(
        kern,
        out_shape=jax.ShapeDtypeStruct((G, N, nclass), jnp.float32),
        grid=(G // gpb,),
        in_specs=in_specs,
        out_specs=pl.BlockSpec((gpb, N, nclass), lambda i: (i, 0, 0)),
        compiler_params=pltpu.CompilerParams(
            dimension_semantics=("parallel",)),
        cost_estimate=cost,
    )(X, S, V, Vt, R3, W1s, W1l, b1, W2s, W2l, b2, Wm, bm,
      filt1_W, filt1_b, filt2_W, filt2_b)


# speedup vs baseline: 1.4731x; 1.0217x over previous
import functools

import jax
import jax.numpy as jnp
from jax import lax
from jax.experimental import pallas as pl
from jax.experimental.pallas import tpu as pltpu

_NCLASS = 64
_LONG = (2, 4)
_GPB = 4


def _ta(lhs, rhs):
    return lax.dot_general(lhs, rhs, (((0,), (0,)), ((), ())),
                           preferred_element_type=jnp.float32)


def _tab(lhs, rhs):
    return lax.dot_general(lhs, rhs, (((0,), (1,)), ((), ())),
                           preferred_element_type=jnp.float32)


def _nn(lhs, rhs):
    return jnp.dot(lhs, rhs, preferred_element_type=jnp.float32)


def _ln_kernel(x_ref, s_ref, v_ref, vt_ref, r_ref,
               w1s_ref, w1l_ref, b1_ref,
               w2s_ref, w2l_ref, b2_ref,
               wm_ref, bm_ref, f1w_ref, f1b_ref, f2w_ref, f2b_ref,
               o_ref, *, n_long, nclass, gpb):
    bf = jnp.bfloat16

    b1c = jnp.transpose(b1_ref[...], (1, 0))
    b2c = jnp.transpose(b2_ref[...], (1, 0))
    bm_row = bm_ref[...]
    f1w, f1b = f1w_ref[...], f1b_ref[...]
    f2w, f2b = f2w_ref[...], f2b_ref[...]

    for g in range(gpb):
        Xg = x_ref[g]
        Sg = s_ref[g]
        Vg = v_ref[g]
        Vtg = vt_ref[g]
        Rv = r_ref[g]

        Rp = {1: Rv}
        cur = Rv
        for t in range(2, max(_LONG) + 1):
            cur = cur * Rv
            if t in _LONG:
                Rp[t] = cur

        def dvec(fw, fb, s):
            d = fb[0:1, s:s + 1]
            for ti, t in enumerate(_LONG):
                d = d + fw[ti:ti + 1, s:s + 1] * Rp[t]
            return d

        A1 = _ta(Xg, Sg)
        A1b = A1.astype(bf)
        A2 = _nn(A1b, Sg)
        A2b = A2.astype(bf)
        U = _ta(Xg, Vg)
        Gt = jnp.concatenate(
            [U * dvec(f1w, f1b, s) for s in range(n_long)], axis=0)
        Pt = _ta(w1l_ref[...], Gt.astype(bf))
        acc = _tab(w1s_ref[0], Xg)
        acc = acc + _ta(w1s_ref[1], A1b)
        acc = acc + _ta(w1s_ref[2], A2b)
        acc = acc + _nn(Pt.astype(bf), Vtg)
        Z1b = jnp.maximum(acc + b1c, 0.0).astype(bf)

        B1 = _nn(Z1b, Sg)
        B1b = B1.astype(bf)
        B2 = _nn(B1b, Sg)
        B2b = B2.astype(bf)
        U2 = _nn(Z1b, Vg)
        Gt2 = jnp.concatenate(
            [U2 * dvec(f2w, f2b, s) for s in range(n_long)], axis=0)
        Pt2 = _ta(w2l_ref[...], Gt2.astype(bf))
        acc2 = _ta(w2s_ref[0], Z1b)
        acc2 = acc2 + _ta(w2s_ref[1], B1b)
        acc2 = acc2 + _ta(w2s_ref[2], B2b)
        acc2 = acc2 + _nn(Pt2.astype(bf), Vtg)
        Z2b = jnp.maximum(acc2 + b2c, 0.0).astype(bf)

        logits = _ta(Z2b, wm_ref[...]) + bm_row
        logits = jnp.maximum(logits, 0.0)
        col = lax.broadcasted_iota(jnp.int32, logits.shape, 1)
        logits = jnp.where(col < nclass, logits, jnp.float32(-1e30))
        m = jnp.max(logits, axis=1, keepdims=True)
        sh = logits - m
        lsm = sh - jnp.log(jnp.sum(jnp.exp(sh), axis=1, keepdims=True))
        o_ref[g] = lsm[:, :nclass]


def kernel(X, S, V, Vt, R, W1s, W1l, b1, W2s, W2l, b2, Wm, bm,
           filt1_W, filt1_b, filt2_W, filt2_b):
    G, N, F0 = X.shape
    Kp = V.shape[-1]
    n_long = len(_LONG)
    nclass = _NCLASS
    F1 = b1.shape[1]
    F2 = b2.shape[1]
    Cpad = Wm.shape[1]
    gpb = _GPB if G % _GPB == 0 else 1

    R3 = R.reshape(G, 1, Kp).astype(jnp.float32)

    kern = functools.partial(_ln_kernel, n_long=n_long, nclass=nclass,
                             gpb=gpb)

    in_specs = [
        pl.BlockSpec((gpb, N, F0), lambda i: (i, 0, 0)),
        pl.BlockSpec((gpb, N, N), lambda i: (i, 0, 0)),
        pl.BlockSpec((gpb, N, Kp), lambda i: (i, 0, 0)),
        pl.BlockSpec((gpb, Kp, N), lambda i: (i, 0, 0)),
        pl.BlockSpec((gpb, 1, Kp), lambda i: (i, 0, 0)),
        pl.BlockSpec(W1s.shape, lambda i: (0, 0, 0)),
        pl.BlockSpec(W1l.shape, lambda i: (0, 0)),
        pl.BlockSpec(b1.shape, lambda i: (0, 0)),
        pl.BlockSpec(W2s.shape, lambda i: (0, 0, 0)),
        pl.BlockSpec(W2l.shape, lambda i: (0, 0)),
        pl.BlockSpec(b2.shape, lambda i: (0, 0)),
        pl.BlockSpec(Wm.shape, lambda i: (0, 0)),
        pl.BlockSpec(bm.shape, lambda i: (0, 0)),
        pl.BlockSpec(filt1_W.shape, lambda i: (0, 0)),
        pl.BlockSpec(filt1_b.shape, lambda i: (0, 0)),
        pl.BlockSpec(filt2_W.shape, lambda i: (0, 0)),
        pl.BlockSpec(filt2_b.shape, lambda i: (0, 0)),
    ]

    flops_layer = G * (2 * 2 * N * N * F1
                       + 2 * 3 * N * F1 * F1
                       + 2 * N * F1 * Kp * 2)
    cost = pl.CostEstimate(
        flops=int(2 * flops_layer + G * 2 * N * F2 * Cpad),
        transcendentals=int(G * N * (Cpad + 1)),
        bytes_accessed=int(X.size * 2 + S.size * 2 + 2 * V.size * 2
                           + G * N * nclass * 4))

    return pl.pallas_call(
        kern,
        out_shape=jax.ShapeDtypeStruct((G, N, nclass), jnp.float32),
        grid=(G // gpb,),
        in_specs=in_specs,
        out_specs=pl.BlockSpec((gpb, N, nclass), lambda i: (i, 0, 0)),
        compiler_params=pltpu.CompilerParams(
            dimension_semantics=("parallel",)),
        cost_estimate=cost,
    )(X, S, V, Vt, R3, W1s, W1l, b1, W2s, W2l, b2, Wm, bm,
      filt1_W, filt1_b, filt2_W, filt2_b)
